# baseline (speedup 1.0000x reference)
import functools

import jax
import jax.numpy as jnp
from jax import lax
from jax.experimental import pallas as pl
from jax.experimental.pallas import tpu as pltpu
from jax.experimental.pallas import tpu_sc as plsc

_B, _S, _H = 1, 2048, 768
_E, _K, _I = 8, 2, 1536
_ALPHA = 1.702
_LIMIT = 7.0
_T = _B * _S
_TB = 512
_NSLOT = _K * _T + _E * _TB
_NP = _K * _T
_NBLK = _NSLOT // _TB + _E
_MAXJ = _T // _TB
_EPC = _E // 2

_NC, _NS = 2, 16
_NW = _NC * _NS
_PW = _NP // _NW


def _router_kernel(x_ref, rw_ref, rb_ref, ltri_ref,
                   scores_ref, w0_ref, w1_ref, d0_ref, d1_ref, meta_ref):
    x = x_ref[...]
    logits = jnp.dot(x.astype(jnp.bfloat16), rw_ref[...].astype(jnp.bfloat16),
                     preferred_element_type=jnp.float32) + rb_ref[...]
    iota = lax.broadcasted_iota(jnp.int32, (_T, _E), 1)
    m0 = jnp.max(logits, axis=1, keepdims=True)
    i0 = jnp.min(jnp.where(logits == m0, iota, _E), axis=1, keepdims=True)
    oh0 = (iota == i0).astype(jnp.float32)
    l1 = jnp.where(iota == i0, -jnp.inf, logits)
    m1 = jnp.max(l1, axis=1, keepdims=True)
    i1 = jnp.min(jnp.where(l1 == m1, iota, _E), axis=1, keepdims=True)
    oh1 = (iota == i1).astype(jnp.float32)
    w0 = jax.nn.sigmoid(m0 - m1)
    w1 = jax.nn.sigmoid(m1 - m0)
    scores_ref[...] = w0 * oh0 + w1 * oh1
    w0_ref[...] = w0
    w1_ref[...] = w1

    mask = (oh0 + oh1).astype(jnp.bfloat16)
    cb = ltri_ref.shape[0]
    cum = jnp.zeros((1, _E), jnp.float32)
    parts = []
    for b in range(_T // cb):
        pb = jnp.dot(ltri_ref[...], mask[b * cb:(b + 1) * cb],
                     preferred_element_type=jnp.float32)
        parts.append(pb + cum)
        cum = cum + pb[cb - 1:cb, :]
    cinc = jnp.concatenate(parts, axis=0)
    totals = cum
    padded = jnp.floor((totals + (_TB - 1)) * (1.0 / _TB)) * _TB
    off = jnp.zeros((1, _E), jnp.float32)
    for s in range(1, _E):
        off = off + jnp.concatenate(
            [jnp.zeros((1, s), jnp.float32), padded[:, :_E - s]], axis=1)
    c0 = jnp.sum(oh0 * cinc, axis=1, keepdims=True)
    c1 = jnp.sum(oh1 * cinc, axis=1, keepdims=True)
    o0 = jnp.sum(oh0 * off, axis=1, keepdims=True)
    o1 = jnp.sum(oh1 * off, axis=1, keepdims=True)
    d0_ref[...] = (o0 + c0 - 1.0).astype(jnp.int32)
    d1_ref[...] = (o1 + c1 - 1.0).astype(jnp.int32)
    meta_ref[...] = jnp.concatenate([off, totals], axis=1).astype(jnp.int32)


def _router_call(x, rw, rb, ltri):
    return pl.pallas_call(
        _router_kernel,
        out_shape=[
            jax.ShapeDtypeStruct((_T, _E), jnp.float32),
            jax.ShapeDtypeStruct((_T, 1), jnp.float32),
            jax.ShapeDtypeStruct((_T, 1), jnp.float32),
            jax.ShapeDtypeStruct((_T, 1), jnp.int32),
            jax.ShapeDtypeStruct((_T, 1), jnp.int32),
            jax.ShapeDtypeStruct((1, 16), jnp.int32),
        ],
    )(x, rw, rb, ltri)


@functools.cache
def _sc_kernels():
    mesh = plsc.VectorSubcoreMesh(core_axis_name="c", subcore_axis_name="s")

    @functools.partial(
        pl.kernel, mesh=mesh,
        out_type=jax.ShapeDtypeStruct((_NBLK * _TB, _H), jnp.float32),
        scratch_types=[
            pltpu.VMEM((_PW,), jnp.int32),
            pltpu.VMEM((_PW, _H), jnp.float32),
            pltpu.SemaphoreType.DMA,
        ],
    )
    def dispatch(x_hbm, dest_hbm, xs_hbm, idx_v, rows_v, sem):
        wid = lax.axis_index("s") * _NC + lax.axis_index("c")
        base = wid * _PW
        tok_base = lax.rem(base, _T)
        pltpu.sync_copy(x_hbm.at[pl.ds(tok_base, _PW)], rows_v)
        pltpu.sync_copy(dest_hbm.at[pl.ds(base, _PW)], idx_v)
        pltpu.async_copy(rows_v, xs_hbm.at[idx_v], sem).wait()

    @functools.partial(
        pl.kernel, mesh=mesh,
        out_type=jax.ShapeDtypeStruct((_NP, _H), jnp.float32),
        scratch_types=[
            pltpu.VMEM((_PW,), jnp.int32),
            pltpu.VMEM((_PW, _H), jnp.float32),
            pltpu.SemaphoreType.DMA,
        ],
    )
    def combine_gather(ys_hbm, dest_hbm, g_hbm, idx_v, rows_v, sem):
        wid = lax.axis_index("s") * _NC + lax.axis_index("c")
        base = wid * _PW
        pltpu.sync_copy(dest_hbm.at[pl.ds(base, _PW)], idx_v)
        pltpu.async_copy(ys_hbm.at[idx_v], rows_v, sem).wait()
        pltpu.sync_copy(rows_v, g_hbm.at[pl.ds(base, _PW)])

    return dispatch, combine_gather


def _sc_dispatch(x, dest):
    return _sc_kernels()[0](x, dest)


def _sc_combine_gather(ys, dest):
    return _sc_kernels()[1](ys, dest)


def _trips(meta_ref, eg):
    cnt = meta_ref[8 + eg]
    return (cnt + (_TB - 1)) // _TB


def _xs_index(i, e, j, meta_ref):
    eg = i * _EPC + e
    trip = _trips(meta_ref, eg)
    blk0 = meta_ref[eg] // _TB
    inact = jnp.where(trip > 0, blk0 + trip - 1, _NSLOT // _TB + eg)
    return jnp.where(j < trip, blk0 + j, inact), 0


def _w_index(i, e, j, meta_ref):
    return i * _EPC + e, 0, 0


def _expert_kernel(meta_ref, xs_ref, gup_ref, gub_ref, dwn_ref, bd_ref,
                   ys_ref, wgs_ref, wds_ref, semg, semd):
    i = pl.program_id(0)
    e = pl.program_id(1)
    j = pl.program_id(2)
    eg = i * _EPC + e

    @pl.when((e == 0) & (j == 0))
    def _():
        for ee in range(_EPC):
            src = i * _EPC + ee
            for q in range(4):
                sl = pl.ds(q * (_I // 2), _I // 2)
                pltpu.make_async_copy(
                    gup_ref.at[src, :, sl], wgs_ref.at[ee, :, sl],
                    semg.at[ee]).start()
            for q in range(2):
                sl = pl.ds(q * (_H // 2), _H // 2)
                pltpu.make_async_copy(
                    dwn_ref.at[src, :, sl], wds_ref.at[ee, :, sl],
                    semd.at[ee]).start()

    @pl.when(j == 0)
    def _():
        pltpu.make_async_copy(gup_ref.at[eg], wgs_ref.at[e], semg.at[e]).wait()
        pltpu.make_async_copy(dwn_ref.at[eg], wds_ref.at[e], semd.at[e]).wait()

    @pl.when(j < _trips(meta_ref, eg))
    def _():
        xt = xs_ref[...].astype(jnp.bfloat16)
        gu = jnp.dot(xt, wgs_ref[e], preferred_element_type=jnp.float32)
        gu = jnp.minimum(gu + gub_ref[0], _LIMIT)
        gate = gu[:, :_I]
        up = jnp.maximum(gu[:, _I:], -_LIMIT)
        glu = gate * jax.nn.sigmoid(gate * _ALPHA)
        act = ((up + 1.0) * glu).astype(jnp.bfloat16)
        ys_ref[...] = jnp.dot(act, wds_ref[e],
                              preferred_element_type=jnp.float32) + bd_ref[0]


def _expert_call(meta, xs, gup, gub, dwn, bd):
    grid_spec = pltpu.PrefetchScalarGridSpec(
        num_scalar_prefetch=1,
        grid=(2, _EPC, _MAXJ),
        in_specs=[
            pl.BlockSpec((_TB, _H), _xs_index),
            pl.BlockSpec(memory_space=pltpu.MemorySpace.HBM),
            pl.BlockSpec((1, 1, 2 * _I), _w_index),
            pl.BlockSpec(memory_space=pltpu.MemorySpace.HBM),
            pl.BlockSpec((1, 1, _H), _w_index),
        ],
        out_specs=pl.BlockSpec((_TB, _H), _xs_index),
        scratch_shapes=[
            pltpu.VMEM((_EPC, _H, 2 * _I), jnp.bfloat16),
            pltpu.VMEM((_EPC, _I, _H), jnp.bfloat16),
            pltpu.SemaphoreType.DMA((_EPC,)),
            pltpu.SemaphoreType.DMA((_EPC,)),
        ],
    )
    return pl.pallas_call(
        _expert_kernel,
        grid_spec=grid_spec,
        out_shape=jax.ShapeDtypeStruct((_NBLK * _TB, _H), jnp.float32),
        compiler_params=pltpu.CompilerParams(
            dimension_semantics=("parallel", "arbitrary", "arbitrary"),
        ),
    )(meta, xs, gup, gub, dwn, bd)


def _combine_kernel(g0_ref, g1_ref, w0_ref, w1_ref, out_ref):
    out_ref[...] = w0_ref[...] * g0_ref[...] + w1_ref[...] * g1_ref[...]


def _combine_call(g, w0, w1):
    nt = 4
    tile = _T // nt
    return pl.pallas_call(
        _combine_kernel,
        grid=(nt,),
        in_specs=[
            pl.BlockSpec((tile, _H), lambda i: (i, 0)),
            pl.BlockSpec((tile, _H), lambda i: (i + nt, 0)),
            pl.BlockSpec((tile, 1), lambda i: (i, 0)),
            pl.BlockSpec((tile, 1), lambda i: (i, 0)),
        ],
        out_specs=pl.BlockSpec((tile, _H), lambda i: (i, 0)),
        out_shape=jax.ShapeDtypeStruct((_T, _H), jnp.float32),
        compiler_params=pltpu.CompilerParams(
            dimension_semantics=("parallel",),
        ),
    )(g, g, w0, w1)


def kernel(hidden_states, router_weight, router_bias, gate_up_proj,
           gate_up_proj_bias, down_proj, down_proj_bias):
    x = hidden_states.reshape(_T, _H)
    rb = router_bias.reshape(1, _E)
    ltri = jnp.tril(jnp.ones((256, 256), jnp.bfloat16))
    scores, w0, w1, d0, d1, meta = _router_call(x, router_weight, rb, ltri)
    dest = jnp.concatenate([d0.reshape(_T), d1.reshape(_T)])
    xs = _sc_dispatch(x, dest)
    gup = gate_up_proj.astype(jnp.bfloat16)
    dwn = down_proj.astype(jnp.bfloat16)
    gub = gate_up_proj_bias.reshape(_E, 1, 2 * _I)
    dnb = down_proj_bias.reshape(_E, 1, _H)
    ys = _expert_call(meta.reshape(16), xs, gup, gub, dwn, dnb)
    g = _sc_combine_gather(ys, dest)
    out = _combine_call(g, w0, w1)
    return out.reshape(_B, _S, _H), scores


# speedup vs baseline: 1.0258x; 1.0258x over previous
import functools

import jax
import jax.numpy as jnp
from jax import lax
from jax.experimental import pallas as pl
from jax.experimental.pallas import tpu as pltpu
from jax.experimental.pallas import tpu_sc as plsc

_B, _S, _H = 1, 2048, 768
_E, _K, _I = 8, 2, 1536
_ALPHA = 1.702
_LIMIT = 7.0
_T = _B * _S
_TB = 256
_NSLOT = _K * _T + _E * _TB
_NP = _K * _T
_NBLK = _NSLOT // _TB + _E
_MAXJ = _T // _TB
_EPC = _E // 2

_NC, _NS = 2, 16
_NW = _NC * _NS
_PW = _NP // _NW


def _router_kernel(x_ref, rw_ref, rb_ref, ltri_ref,
                   scores_ref, w0_ref, w1_ref, d0_ref, d1_ref, meta_ref):
    x = x_ref[...]
    logits = jnp.dot(x.astype(jnp.bfloat16), rw_ref[...].astype(jnp.bfloat16),
                     preferred_element_type=jnp.float32) + rb_ref[...]
    iota = lax.broadcasted_iota(jnp.int32, (_T, _E), 1)
    m0 = jnp.max(logits, axis=1, keepdims=True)
    i0 = jnp.min(jnp.where(logits == m0, iota, _E), axis=1, keepdims=True)
    oh0 = (iota == i0).astype(jnp.float32)
    l1 = jnp.where(iota == i0, -jnp.inf, logits)
    m1 = jnp.max(l1, axis=1, keepdims=True)
    i1 = jnp.min(jnp.where(l1 == m1, iota, _E), axis=1, keepdims=True)
    oh1 = (iota == i1).astype(jnp.float32)
    w0 = jax.nn.sigmoid(m0 - m1)
    w1 = jax.nn.sigmoid(m1 - m0)
    scores_ref[...] = w0 * oh0 + w1 * oh1
    w0_ref[...] = w0
    w1_ref[...] = w1

    mask = (oh0 + oh1).astype(jnp.bfloat16)
    cb = ltri_ref.shape[0]
    cum = jnp.zeros((1, _E), jnp.float32)
    parts = []
    for b in range(_T // cb):
        pb = jnp.dot(ltri_ref[...], mask[b * cb:(b + 1) * cb],
                     preferred_element_type=jnp.float32)
        parts.append(pb + cum)
        cum = cum + pb[cb - 1:cb, :]
    cinc = jnp.concatenate(parts, axis=0)
    totals = cum
    padded = jnp.floor((totals + (_TB - 1)) * (1.0 / _TB)) * _TB
    off = jnp.zeros((1, _E), jnp.float32)
    for s in range(1, _E):
        off = off + jnp.concatenate(
            [jnp.zeros((1, s), jnp.float32), padded[:, :_E - s]], axis=1)
    c0 = jnp.sum(oh0 * cinc, axis=1, keepdims=True)
    c1 = jnp.sum(oh1 * cinc, axis=1, keepdims=True)
    o0 = jnp.sum(oh0 * off, axis=1, keepdims=True)
    o1 = jnp.sum(oh1 * off, axis=1, keepdims=True)
    d0_ref[...] = (o0 + c0 - 1.0).astype(jnp.int32)
    d1_ref[...] = (o1 + c1 - 1.0).astype(jnp.int32)
    meta_ref[...] = jnp.concatenate([off, totals], axis=1).astype(jnp.int32)


def _router_call(x, rw, rb, ltri):
    return pl.pallas_call(
        _router_kernel,
        out_shape=[
            jax.ShapeDtypeStruct((_T, _E), jnp.float32),
            jax.ShapeDtypeStruct((_T, 1), jnp.float32),
            jax.ShapeDtypeStruct((_T, 1), jnp.float32),
            jax.ShapeDtypeStruct((_T, 1), jnp.int32),
            jax.ShapeDtypeStruct((_T, 1), jnp.int32),
            jax.ShapeDtypeStruct((1, 16), jnp.int32),
        ],
    )(x, rw, rb, ltri)


@functools.cache
def _sc_kernels():
    mesh = plsc.VectorSubcoreMesh(core_axis_name="c", subcore_axis_name="s")

    @functools.partial(
        pl.kernel, mesh=mesh,
        out_type=jax.ShapeDtypeStruct((_NBLK * _TB, _H), jnp.float32),
        scratch_types=[
            pltpu.VMEM((_PW,), jnp.int32),
            pltpu.VMEM((_PW, _H), jnp.float32),
            pltpu.SemaphoreType.DMA,
        ],
    )
    def dispatch(x_hbm, dest_hbm, xs_hbm, idx_v, rows_v, sem):
        wid = lax.axis_index("s") * _NC + lax.axis_index("c")
        base = wid * _PW
        tok_base = lax.rem(base, _T)
        pltpu.sync_copy(x_hbm.at[pl.ds(tok_base, _PW)], rows_v)
        pltpu.sync_copy(dest_hbm.at[pl.ds(base, _PW)], idx_v)
        pltpu.async_copy(rows_v, xs_hbm.at[idx_v], sem).wait()

    @functools.partial(
        pl.kernel, mesh=mesh,
        out_type=jax.ShapeDtypeStruct((_NP, _H), jnp.float32),
        scratch_types=[
            pltpu.VMEM((_PW,), jnp.int32),
            pltpu.VMEM((_PW, _H), jnp.float32),
            pltpu.SemaphoreType.DMA,
        ],
    )
    def combine_gather(ys_hbm, dest_hbm, g_hbm, idx_v, rows_v, sem):
        wid = lax.axis_index("s") * _NC + lax.axis_index("c")
        base = wid * _PW
        pltpu.sync_copy(dest_hbm.at[pl.ds(base, _PW)], idx_v)
        pltpu.async_copy(ys_hbm.at[idx_v], rows_v, sem).wait()
        pltpu.sync_copy(rows_v, g_hbm.at[pl.ds(base, _PW)])

    return dispatch, combine_gather


def _sc_dispatch(x, dest):
    return _sc_kernels()[0](x, dest)


def _sc_combine_gather(ys, dest):
    return _sc_kernels()[1](ys, dest)


def _trips(meta_ref, eg):
    cnt = meta_ref[8 + eg]
    return (cnt + (_TB - 1)) // _TB


def _xs_index(i, e, j, meta_ref):
    eg = i * _EPC + e
    trip = _trips(meta_ref, eg)
    blk0 = meta_ref[eg] // _TB
    inact = jnp.where(trip > 0, blk0 + trip - 1, _NSLOT // _TB + eg)
    return jnp.where(j < trip, blk0 + j, inact), 0


def _w_index(i, e, j, meta_ref):
    return i * _EPC + e, 0, 0


def _expert_kernel(meta_ref, xs_ref, gup_ref, gub_ref, dwn_ref, bd_ref,
                   ys_ref, wgs_ref, wds_ref, semg, semd):
    i = pl.program_id(0)
    e = pl.program_id(1)
    j = pl.program_id(2)
    eg = i * _EPC + e

    @pl.when((e == 0) & (j == 0))
    def _():
        for ee in range(_EPC):
            src = i * _EPC + ee
            pltpu.make_async_copy(
                gup_ref.at[src], wgs_ref.at[ee], semg.at[ee]).start()
            pltpu.make_async_copy(
                dwn_ref.at[src], wds_ref.at[ee], semd.at[ee]).start()

    @pl.when(j == 0)
    def _():
        pltpu.make_async_copy(gup_ref.at[eg], wgs_ref.at[e], semg.at[e]).wait()
        pltpu.make_async_copy(dwn_ref.at[eg], wds_ref.at[e], semd.at[e]).wait()

    @pl.when(j < _trips(meta_ref, eg))
    def _():
        xt = xs_ref[...].astype(jnp.bfloat16)
        gu = jnp.dot(xt, wgs_ref[e], preferred_element_type=jnp.float32)
        gu = jnp.minimum(gu + gub_ref[0], _LIMIT)
        gate = gu[:, :_I]
        up = jnp.maximum(gu[:, _I:], -_LIMIT)
        glu = gate * jax.nn.sigmoid(gate * _ALPHA)
        act = ((up + 1.0) * glu).astype(jnp.bfloat16)
        ys_ref[...] = jnp.dot(act, wds_ref[e],
                              preferred_element_type=jnp.float32) + bd_ref[0]


def _expert_call(meta, xs, gup, gub, dwn, bd):
    grid_spec = pltpu.PrefetchScalarGridSpec(
        num_scalar_prefetch=1,
        grid=(2, _EPC, _MAXJ),
        in_specs=[
            pl.BlockSpec((_TB, _H), _xs_index),
            pl.BlockSpec(memory_space=pltpu.MemorySpace.HBM),
            pl.BlockSpec((1, 1, 2 * _I), _w_index),
            pl.BlockSpec(memory_space=pltpu.MemorySpace.HBM),
            pl.BlockSpec((1, 1, _H), _w_index),
        ],
        out_specs=pl.BlockSpec((_TB, _H), _xs_index),
        scratch_shapes=[
            pltpu.VMEM((_EPC, _H, 2 * _I), jnp.bfloat16),
            pltpu.VMEM((_EPC, _I, _H), jnp.bfloat16),
            pltpu.SemaphoreType.DMA((_EPC,)),
            pltpu.SemaphoreType.DMA((_EPC,)),
        ],
    )
    return pl.pallas_call(
        _expert_kernel,
        grid_spec=grid_spec,
        out_shape=jax.ShapeDtypeStruct((_NBLK * _TB, _H), jnp.float32),
        compiler_params=pltpu.CompilerParams(
            dimension_semantics=("parallel", "arbitrary", "arbitrary"),
        ),
    )(meta, xs, gup, gub, dwn, bd)


def _combine_kernel(g0_ref, g1_ref, w0_ref, w1_ref, out_ref):
    out_ref[...] = w0_ref[...] * g0_ref[...] + w1_ref[...] * g1_ref[...]


def _combine_call(g, w0, w1):
    nt = 4
    tile = _T // nt
    return pl.pallas_call(
        _combine_kernel,
        grid=(nt,),
        in_specs=[
            pl.BlockSpec((tile, _H), lambda i: (i, 0)),
            pl.BlockSpec((tile, _H), lambda i: (i + nt, 0)),
            pl.BlockSpec((tile, 1), lambda i: (i, 0)),
            pl.BlockSpec((tile, 1), lambda i: (i, 0)),
        ],
        out_specs=pl.BlockSpec((tile, _H), lambda i: (i, 0)),
        out_shape=jax.ShapeDtypeStruct((_T, _H), jnp.float32),
        compiler_params=pltpu.CompilerParams(
            dimension_semantics=("parallel",),
        ),
    )(g, g, w0, w1)


def kernel(hidden_states, router_weight, router_bias, gate_up_proj,
           gate_up_proj_bias, down_proj, down_proj_bias):
    x = hidden_states.reshape(_T, _H)
    rb = router_bias.reshape(1, _E)
    ltri = jnp.tril(jnp.ones((256, 256), jnp.bfloat16))
    scores, w0, w1, d0, d1, meta = _router_call(x, router_weight, rb, ltri)
    dest = jnp.concatenate([d0.reshape(_T), d1.reshape(_T)])
    xs = _sc_dispatch(x, dest)
    gup = gate_up_proj.astype(jnp.bfloat16)
    dwn = down_proj.astype(jnp.bfloat16)
    gub = gate_up_proj_bias.reshape(_E, 1, 2 * _I)
    dnb = down_proj_bias.reshape(_E, 1, _H)
    ys = _expert_call(meta.reshape(16), xs, gup, gub, dwn, dnb)
    g = _sc_combine_gather(ys, dest)
    out = _combine_call(g, w0, w1)
    return out.reshape(_B, _S, _H), scores
